# Initial kernel scaffold; baseline (speedup 1.0000x reference)
import jax
import jax.numpy as jnp
from jax.experimental import pallas as pl


def kernel(x, edge_index, W):
    raise NotImplementedError("write your pallas kernel here")


# speedup vs baseline: 5.7591x; 5.7591x over previous
import functools

import jax
import jax.numpy as jnp
from jax import lax
from jax.experimental import pallas as pl
from jax.experimental.pallas import tpu as pltpu
from jax.experimental.pallas import tpu_sc as plsc

N = 10000
E = 320000
D = 128
DP = 144

NC = 2
NS = 16
NW = NC * NS
EPW = E // NW
CHUNK = 80
NCHUNK = EPW // CHUNK

ZROWS = 80
RPS = 640


def _sc_agg_body(xp_hbm, src_hbm, dst_hbm, out_hbm,
                 acc_sh, src_v, dst_v, rows_v, zero_v, sem):
    c = lax.axis_index("c")
    s = lax.axis_index("s")
    wid = c * NS + s

    zvec = jnp.zeros((16,), jnp.float32)

    def zero_row(r, _):
        for j in range(DP // 16):
            zero_v[r, pl.ds(j * 16, 16)] = zvec
        return 0

    lax.fori_loop(0, ZROWS, zero_row, 0)
    for z in range(RPS // ZROWS):
        if z < 5:
            pltpu.sync_copy(zero_v,
                            acc_sh.at[pl.ds(s * RPS + z * ZROWS, ZROWS)])
        else:
            @pl.when(s < NS - 1)
            def _():
                pltpu.sync_copy(zero_v,
                                acc_sh.at[pl.ds(s * RPS + z * ZROWS, ZROWS)])
    plsc.subcore_barrier()

    def body(k, _):
        base = wid * EPW + k * CHUNK
        pltpu.sync_copy(src_hbm.at[pl.ds(base, CHUNK)], src_v)
        pltpu.sync_copy(dst_hbm.at[pl.ds(base, CHUNK)], dst_v)
        pltpu.async_copy(xp_hbm.at[src_v], rows_v, sem).wait()
        pltpu.sync_copy(rows_v, acc_sh.at[dst_v], add=True)
        return 0

    lax.fori_loop(0, NCHUNK, body, 0)
    plsc.subcore_barrier()

    for z in range(RPS // ZROWS):
        r0 = s * RPS + z * ZROWS
        if z < 5:
            pltpu.sync_copy(acc_sh.at[pl.ds(r0, ZROWS)],
                            out_hbm.at[c, pl.ds(r0, ZROWS)])
        else:
            @pl.when(s < NS - 1)
            def _():
                pltpu.sync_copy(acc_sh.at[pl.ds(r0, ZROWS)],
                                out_hbm.at[c, pl.ds(r0, ZROWS)])


_sc_agg = functools.partial(
    pl.kernel,
    out_type=jax.ShapeDtypeStruct((NC, N, DP), jnp.float32),
    mesh=plsc.VectorSubcoreMesh(core_axis_name="c", subcore_axis_name="s"),
    scratch_types=[
        pltpu.VMEM_SHARED((N, DP), jnp.float32),
        pltpu.VMEM((CHUNK,), jnp.int32),
        pltpu.VMEM((CHUNK,), jnp.int32),
        pltpu.VMEM((CHUNK, DP), jnp.float32),
        pltpu.VMEM((ZROWS, DP), jnp.float32),
        pltpu.SemaphoreType.DMA,
    ],
    compiler_params=pltpu.CompilerParams(use_tc_tiling_on_sc=False),
)(_sc_agg_body)


BLK = 1000


def _tc_body(p_ref, w_ref, o_ref):
    p = p_ref[...]
    t = p[0] + p[1]
    deg = t[:, D:D + 1]
    h = t[:, :D] / jnp.clip(deg, 1.0, None)
    o_ref[...] = jnp.dot(h, w_ref[...], preferred_element_type=jnp.float32)


def kernel(x, edge_index, W):
    src = edge_index[0]
    dst = edge_index[1]
    xp = jnp.concatenate(
        [x, jnp.ones((N, 1), x.dtype), jnp.zeros((N, DP - D - 1), x.dtype)],
        axis=1)
    partial = _sc_agg(xp, src, dst)
    z = pl.pallas_call(
        _tc_body,
        grid=(N // BLK,),
        in_specs=[
            pl.BlockSpec((NC, BLK, DP), lambda i: (0, i, 0)),
            pl.BlockSpec((D, D), lambda i: (0, 0)),
        ],
        out_specs=pl.BlockSpec((BLK, D), lambda i: (i, 0)),
        out_shape=jax.ShapeDtypeStruct((N, D), jnp.float32),
    )(partial, W)
    return z


# speedup vs baseline: 9.4762x; 1.6454x over previous
import functools

import jax
import jax.numpy as jnp
from jax import lax
from jax.experimental import pallas as pl
from jax.experimental.pallas import tpu as pltpu
from jax.experimental.pallas import tpu_sc as plsc

N = 10000
E = 320000
D = 128
DP = 144

NC = 2
NS = 16
NW = NC * NS
EPW = E // NW
CHUNK = 40
NCHUNK = EPW // CHUNK

ZROWS = 80
RPS = 640
ZROWS2 = 40


def _sc_agg_body(xp_hbm, src_hbm, dst_hbm, out_hbm,
                 acc_sh, src_v, dst_v, rows0_v, rows1_v, zero_v,
                 sem0, sem1):
    c = lax.axis_index("c")
    s = lax.axis_index("s")
    wid = c * NS + s

    pltpu.sync_copy(src_hbm.at[wid], src_v)
    pltpu.sync_copy(dst_hbm.at[wid], dst_v)

    zvec = jnp.zeros((16,), jnp.float32)

    def zero_row(r, _):
        for j in range(DP // 16):
            zero_v[r, pl.ds(j * 16, 16)] = zvec
        return 0

    lax.fori_loop(0, ZROWS2, zero_row, 0)
    for z in range(RPS // ZROWS2):
        if z < 10:
            pltpu.sync_copy(zero_v,
                            acc_sh.at[pl.ds(s * RPS + z * ZROWS2, ZROWS2)])
        else:
            @pl.when(s < NS - 1)
            def _():
                pltpu.sync_copy(zero_v,
                                acc_sh.at[pl.ds(s * RPS + z * ZROWS2, ZROWS2)])
    plsc.subcore_barrier()

    def start_gather(k, buf, sem):
        pltpu.async_copy(xp_hbm.at[src_v.at[k]], buf, sem)

    def wait_gather(k, buf, sem):
        pltpu.make_async_copy(xp_hbm.at[src_v.at[k]], buf, sem).wait()

    def scatter(k, buf):
        pltpu.sync_copy(buf, acc_sh.at[dst_v.at[k]], add=True)

    start_gather(0, rows0_v, sem0)

    def pair_body(p, _):
        k0 = 2 * p
        start_gather(k0 + 1, rows1_v, sem1)
        wait_gather(k0, rows0_v, sem0)
        scatter(k0, rows0_v)

        @pl.when(k0 + 2 < NCHUNK)
        def _():
            start_gather(k0 + 2, rows0_v, sem0)

        wait_gather(k0 + 1, rows1_v, sem1)
        scatter(k0 + 1, rows1_v)
        return 0

    lax.fori_loop(0, NCHUNK // 2, pair_body, 0)
    plsc.subcore_barrier()

    for z in range(RPS // ZROWS):
        r0 = s * RPS + z * ZROWS
        if z < 5:
            pltpu.sync_copy(acc_sh.at[pl.ds(r0, ZROWS)],
                            out_hbm.at[c, pl.ds(r0, ZROWS)])
        else:
            @pl.when(s < NS - 1)
            def _():
                pltpu.sync_copy(acc_sh.at[pl.ds(r0, ZROWS)],
                                out_hbm.at[c, pl.ds(r0, ZROWS)])


_sc_agg = functools.partial(
    pl.kernel,
    out_type=jax.ShapeDtypeStruct((NC, N, DP), jnp.float32),
    mesh=plsc.VectorSubcoreMesh(core_axis_name="c", subcore_axis_name="s"),
    scratch_types=[
        pltpu.VMEM_SHARED((N, DP), jnp.float32),
        pltpu.VMEM((NCHUNK, CHUNK), jnp.int32),
        pltpu.VMEM((NCHUNK, CHUNK), jnp.int32),
        pltpu.VMEM((CHUNK, DP), jnp.float32),
        pltpu.VMEM((CHUNK, DP), jnp.float32),
        pltpu.VMEM((ZROWS2, DP), jnp.float32),
        pltpu.SemaphoreType.DMA,
        pltpu.SemaphoreType.DMA,
    ],
    compiler_params=pltpu.CompilerParams(use_tc_tiling_on_sc=False),
)(_sc_agg_body)


BLK = 1000


def _tc_body(p_ref, w_ref, o_ref):
    p = p_ref[...]
    t = p[0] + p[1]
    deg = t[:, D:D + 1]
    h = t[:, :D] / jnp.clip(deg, 1.0, None)
    o_ref[...] = jnp.dot(h, w_ref[...], preferred_element_type=jnp.float32)


def kernel(x, edge_index, W):
    src = edge_index[0].reshape(NW, NCHUNK, CHUNK)
    dst = edge_index[1].reshape(NW, NCHUNK, CHUNK)
    xp = jnp.concatenate(
        [x, jnp.ones((N, 1), x.dtype), jnp.zeros((N, DP - D - 1), x.dtype)],
        axis=1)
    partial = _sc_agg(xp, src, dst)
    z = pl.pallas_call(
        _tc_body,
        grid=(N // BLK,),
        in_specs=[
            pl.BlockSpec((NC, BLK, DP), lambda i: (0, i, 0)),
            pl.BlockSpec((D, D), lambda i: (0, 0)),
        ],
        out_specs=pl.BlockSpec((BLK, D), lambda i: (i, 0)),
        out_shape=jax.ShapeDtypeStruct((N, D), jnp.float32),
    )(partial, W)
    return z
